# Initial kernel scaffold; baseline (speedup 1.0000x reference)
import jax
import jax.numpy as jnp
from jax.experimental import pallas as pl


def kernel(points, features, w0_0, b0_0, w0_1, b0_1, w0_2, b0_2, w1_0, b1_0, w1_1, b1_1, w1_2, b1_2, w_agg, b_agg):
    raise NotImplementedError("write your pallas kernel here")


# speedup vs baseline: 1.3867x; 1.3867x over previous
import functools

import jax
import jax.numpy as jnp
from jax.experimental import pallas as pl
from jax.experimental.pallas import tpu as pltpu

NB = 4
NPT = 8192
NSMP = 1024
NROW, NCOL = 8, 1024
RADII2 = (0.2 * 0.2, 0.4 * 0.4)
NNBR = (16, 32)
CPRE = 32


PROW, PCOL = 8, 128


def _fps_body(xs_ref, ys_ref, zs_ref, out_ref):
    row = jax.lax.broadcasted_iota(jnp.int32, (NROW, NCOL), 0)
    col = jax.lax.broadcasted_iota(jnp.int32, (NROW, NCOL), 1)
    flat_idx = row * NCOL + col
    prow = jax.lax.broadcasted_iota(jnp.int32, (PROW, PCOL), 0)
    pcol = jax.lax.broadcasted_iota(jnp.int32, (PROW, PCOL), 1)

    def step(i, carry):
        coords, dists, planes = carry
        pm = (prow == i // PCOL) & (pcol == i % PCOL)
        new_coords, new_dists, new_planes = [], [], []
        for b in range(NB):
            cx, cy, cz = coords[b]
            x = xs_ref[b]
            y = ys_ref[b]
            z = zs_ref[b]
            dx = x - cx
            dy = y - cy
            dz = z - cz
            d = dx * dx + dy * dy + dz * dz
            nd = jnp.minimum(dists[b], d)
            px, py, pz = planes[b]
            new_planes.append((jnp.where(pm, cx, px),
                               jnp.where(pm, cy, py),
                               jnp.where(pm, cz, pz)))
            m = jnp.max(nd)
            far_new = jnp.min(jnp.where(nd == m, flat_idx, NPT))
            eq = flat_idx == far_new
            zero = jnp.float32(0)
            new_coords.append((jnp.sum(jnp.where(eq, x, zero)),
                               jnp.sum(jnp.where(eq, y, zero)),
                               jnp.sum(jnp.where(eq, z, zero))))
            new_dists.append(nd)
        return tuple(new_coords), tuple(new_dists), tuple(new_planes)

    coords0 = tuple((jnp.sum(xs_ref[b, pl.ds(0, 1), pl.ds(0, 1)]),
                     jnp.sum(ys_ref[b, pl.ds(0, 1), pl.ds(0, 1)]),
                     jnp.sum(zs_ref[b, pl.ds(0, 1), pl.ds(0, 1)]))
                    for b in range(NB))
    dists0 = tuple(jnp.full((NROW, NCOL), 1e10, jnp.float32) for _ in range(NB))
    zplane = jnp.zeros((PROW, PCOL), jnp.float32)
    planes0 = tuple((zplane, zplane, zplane) for _ in range(NB))
    _, _, planes = jax.lax.fori_loop(0, NSMP, step, (coords0, dists0, planes0))
    for b in range(NB):
        for c in range(3):
            out_ref[b, c] = planes[b][c]


def _fps_call(xs, ys, zs):
    out = pl.pallas_call(
        _fps_body,
        out_shape=jax.ShapeDtypeStruct((NB, 3, PROW, PCOL), jnp.float32),
    )(xs, ys, zs)
    return out.reshape(NB, 3, NSMP)


def _pre_body(pts_ref, feat_ref, w0_ref, w1_ref, pre0_ref, pre1_ref):
    p = pts_ref[0]
    f = feat_ref[0]
    h = jnp.concatenate([p, f], axis=1)
    for w_ref, o_ref in ((w0_ref, pre0_ref), (w1_ref, pre1_ref)):
        w = w_ref[...]
        o_ref[0] = jax.lax.dot_general(
            h, w, (((1,), (1,)), ((), ())),
            preferred_element_type=jnp.float32)


def _pre_call(points, feats_t, w0, w1):
    return pl.pallas_call(
        _pre_body,
        grid=(NB,),
        in_specs=[
            pl.BlockSpec((1, NPT, 3), lambda b: (b, 0, 0)),
            pl.BlockSpec((1, NPT, 16), lambda b: (b, 0, 0)),
            pl.BlockSpec((CPRE, 19), lambda b: (0, 0)),
            pl.BlockSpec((CPRE, 19), lambda b: (0, 0)),
        ],
        out_specs=[
            pl.BlockSpec((1, NPT, CPRE), lambda b: (b, 0, 0)),
            pl.BlockSpec((1, NPT, CPRE), lambda b: (b, 0, 0)),
        ],
        out_shape=[
            jax.ShapeDtypeStruct((NB, NPT, CPRE), jnp.float32),
            jax.ShapeDtypeStruct((NB, NPT, CPRE), jnp.float32),
        ],
    )(points, feats_t, w0, w1)


KBLK = 256


def _mlp_body(g0_ref, g1_ref, s_ref,
              w00_ref, b00_ref, w01_ref, b01_ref, w02_ref, b02_ref,
              w10_ref, b10_ref, w11_ref, b11_ref, w12_ref, b12_ref,
              wagg_ref, bagg_ref, out_ref):
    c3 = s_ref[0]
    accs = []
    scale_refs = (
        (g0_ref, NNBR[0], w00_ref, b00_ref, w01_ref, b01_ref, w02_ref, b02_ref),
        (g1_ref, NNBR[1], w10_ref, b10_ref, w11_ref, b11_ref, w12_ref, b12_ref),
    )
    for g_ref, nn, w0_ref, b0_ref, w1_ref, b1_ref, w2_ref, b2_ref in scale_refs:
        g = g_ref[0]
        wx = w0_ref[...][:, :3]
        ct = jax.lax.dot_general(
            c3, wx, (((0,), (1,)), ((), ())),
            preferred_element_type=jnp.float32)
        bct = b0_ref[...] - ct
        h1 = jnp.maximum(g + bct[:, None, :], 0.0)
        h1f = h1.reshape(KBLK * nn, CPRE)
        h2 = jnp.maximum(
            jax.lax.dot_general(h1f, w1_ref[...], (((1,), (1,)), ((), ())),
                                preferred_element_type=jnp.float32)
            + b1_ref[...], 0.0)
        h3 = jnp.maximum(
            jax.lax.dot_general(h2, w2_ref[...], (((1,), (1,)), ((), ())),
                                preferred_element_type=jnp.float32)
            + b2_ref[...], 0.0)
        accs.append(jnp.max(h3.reshape(KBLK, nn, 64), axis=1))
    cat = jnp.concatenate(accs, axis=1)
    out = jnp.maximum(
        jax.lax.dot_general(cat, wagg_ref[...], (((1,), (1,)), ((), ())),
                            preferred_element_type=jnp.float32)
        + bagg_ref[...], 0.0)
    out_ref[0] = out


def _mlp_call(g0, g1, sxyz, w00, b00, w01, b01, w02, b02,
              w10, b10, w11, b11, w12, b12, wagg, bagg):
    full = lambda shape: pl.BlockSpec(shape, lambda b, kb: tuple(0 for _ in shape))
    return pl.pallas_call(
        _mlp_body,
        grid=(NB, NSMP // KBLK),
        in_specs=[
            pl.BlockSpec((1, KBLK, NNBR[0], CPRE), lambda b, kb: (b, kb, 0, 0)),
            pl.BlockSpec((1, KBLK, NNBR[1], CPRE), lambda b, kb: (b, kb, 0, 0)),
            pl.BlockSpec((1, 3, KBLK), lambda b, kb: (b, 0, kb)),
            full((32, 19)), full((1, 32)),
            full((32, 32)), full((1, 32)),
            full((64, 32)), full((1, 64)),
            full((32, 19)), full((1, 32)),
            full((32, 32)), full((1, 32)),
            full((64, 32)), full((1, 64)),
            full((64, 128)), full((1, 64)),
        ],
        out_specs=pl.BlockSpec((1, KBLK, 64), lambda b, kb: (b, kb, 0)),
        out_shape=jax.ShapeDtypeStruct((NB, NSMP, 64), jnp.float32),
    )(g0, g1, sxyz, w00, b00, w01, b01, w02, b02,
      w10, b10, w11, b11, w12, b12, wagg, bagg)


def _bq_jnp(sampled, points, radius, nn):
    n = points.shape[1]
    d2 = jnp.sum((sampled[:, :, None, :] - points[:, None, :, :]) ** 2, axis=-1)
    mask = d2 < radius
    idx = jnp.where(mask, jnp.arange(n, dtype=jnp.int32)[None, None, :], n)
    idx = jnp.sort(idx, axis=-1)[:, :, :nn]
    first = idx[:, :, :1]
    idx = jnp.where(idx == n, first, idx)
    return jnp.minimum(idx, n - 1)


def kernel(points, features, w0_0, b0_0, w0_1, b0_1, w0_2, b0_2,
           w1_0, b1_0, w1_1, b1_1, w1_2, b1_2, w_agg, b_agg):
    pts_r = points.transpose(0, 2, 1).reshape(NB, 3, NROW, NCOL)
    xs, ys, zs = pts_r[:, 0], pts_r[:, 1], pts_r[:, 2]
    sxyz = _fps_call(xs, ys, zs)
    sampled = sxyz.transpose(0, 2, 1)

    feats_t = features.transpose(0, 2, 1)
    pre0, pre1 = _pre_call(points, feats_t, w0_0, w1_0)

    idx0 = _bq_jnp(sampled, points, RADII2[0], NNBR[0])
    idx1 = _bq_jnp(sampled, points, RADII2[1], NNBR[1])
    g0 = jnp.take_along_axis(pre0, idx0.reshape(NB, -1)[:, :, None], axis=1)
    g0 = g0.reshape(NB, NSMP, NNBR[0], CPRE)
    g1 = jnp.take_along_axis(pre1, idx1.reshape(NB, -1)[:, :, None], axis=1)
    g1 = g1.reshape(NB, NSMP, NNBR[1], CPRE)

    feats_out = _mlp_call(
        g0, g1, sxyz,
        w0_0, b0_0.reshape(1, -1), w0_1, b0_1.reshape(1, -1),
        w0_2, b0_2.reshape(1, -1),
        w1_0, b1_0.reshape(1, -1), w1_1, b1_1.reshape(1, -1),
        w1_2, b1_2.reshape(1, -1),
        w_agg, b_agg.reshape(1, -1))
    return sampled, feats_out.transpose(0, 2, 1)


# speedup vs baseline: 14.9587x; 10.7875x over previous
import functools

import jax
import jax.numpy as jnp
from jax.experimental import pallas as pl
from jax.experimental.pallas import tpu as pltpu

NB = 4
NPT = 8192
NSMP = 1024
NROW, NCOL = 8, 1024
RADII2 = (0.2 * 0.2, 0.4 * 0.4)
NNBR = (16, 32)
CPRE = 32


PROW, PCOL = 8, 128


def _fps_body(xs_ref, ys_ref, zs_ref, out_ref):
    row = jax.lax.broadcasted_iota(jnp.int32, (NROW, NCOL), 0)
    col = jax.lax.broadcasted_iota(jnp.int32, (NROW, NCOL), 1)
    flat_idx = row * NCOL + col
    prow = jax.lax.broadcasted_iota(jnp.int32, (PROW, PCOL), 0)
    pcol = jax.lax.broadcasted_iota(jnp.int32, (PROW, PCOL), 1)

    def step(i, carry):
        coords, dists, planes = carry
        pm = (prow == i // PCOL) & (pcol == i % PCOL)
        new_coords, new_dists, new_planes = [], [], []
        for b in range(NB):
            cx, cy, cz = coords[b]
            x = xs_ref[b]
            y = ys_ref[b]
            z = zs_ref[b]
            dx = x - cx
            dy = y - cy
            dz = z - cz
            d = dx * dx + dy * dy + dz * dz
            nd = jnp.minimum(dists[b], d)
            px, py, pz = planes[b]
            new_planes.append((jnp.where(pm, cx, px),
                               jnp.where(pm, cy, py),
                               jnp.where(pm, cz, pz)))
            m = jnp.max(nd)
            far_new = jnp.min(jnp.where(nd == m, flat_idx, NPT))
            eq = flat_idx == far_new
            zero = jnp.float32(0)
            new_coords.append((jnp.sum(jnp.where(eq, x, zero)),
                               jnp.sum(jnp.where(eq, y, zero)),
                               jnp.sum(jnp.where(eq, z, zero))))
            new_dists.append(nd)
        return tuple(new_coords), tuple(new_dists), tuple(new_planes)

    coords0 = tuple((jnp.sum(xs_ref[b, pl.ds(0, 1), pl.ds(0, 1)]),
                     jnp.sum(ys_ref[b, pl.ds(0, 1), pl.ds(0, 1)]),
                     jnp.sum(zs_ref[b, pl.ds(0, 1), pl.ds(0, 1)]))
                    for b in range(NB))
    dists0 = tuple(jnp.full((NROW, NCOL), 1e10, jnp.float32) for _ in range(NB))
    zplane = jnp.zeros((PROW, PCOL), jnp.float32)
    planes0 = tuple((zplane, zplane, zplane) for _ in range(NB))
    _, _, planes = jax.lax.fori_loop(0, NSMP, step, (coords0, dists0, planes0))
    for b in range(NB):
        for c in range(3):
            out_ref[b, c] = planes[b][c]


def _fps_call(xs, ys, zs):
    out = pl.pallas_call(
        _fps_body,
        out_shape=jax.ShapeDtypeStruct((NB, 3, PROW, PCOL), jnp.float32),
    )(xs, ys, zs)
    return out.reshape(NB, 3, NSMP)


CROW = 128


def _pre_body(pts_ref, feat_ref, w0_ref, w1_ref, pre_ref):
    p = pts_ref[0]
    f = feat_ref[0]
    h = jnp.concatenate([p, f], axis=1)
    pre0 = jax.lax.dot_general(h, w0_ref[...], (((1,), (1,)), ((), ())),
                               preferred_element_type=jnp.float32)
    pre1 = jax.lax.dot_general(h, w1_ref[...], (((1,), (1,)), ((), ())),
                               preferred_element_type=jnp.float32)
    pre_ref[0, :, 0:CPRE] = pre0
    pre_ref[0, :, CPRE:2 * CPRE] = pre1


def _pre_call(points, feats_t, w0, w1):
    return pl.pallas_call(
        _pre_body,
        grid=(NB,),
        in_specs=[
            pl.BlockSpec((1, NPT, 3), lambda b: (b, 0, 0)),
            pl.BlockSpec((1, NPT, 16), lambda b: (b, 0, 0)),
            pl.BlockSpec((CPRE, 19), lambda b: (0, 0)),
            pl.BlockSpec((CPRE, 19), lambda b: (0, 0)),
        ],
        out_specs=pl.BlockSpec((1, NPT, CROW), lambda b: (b, 0, 0)),
        out_shape=jax.ShapeDtypeStruct((NB, NPT, CROW), jnp.float32),
    )(points, feats_t, w0, w1)


KBLK = 256


def _mlp_body(g0_ref, g1_ref, s_ref,
              w00_ref, b00_ref, w01_ref, b01_ref, w02_ref, b02_ref,
              w10_ref, b10_ref, w11_ref, b11_ref, w12_ref, b12_ref,
              wagg_ref, bagg_ref, out_ref):
    c3 = s_ref[0]
    accs = []
    scale_refs = (
        (g0_ref, NNBR[0], w00_ref, b00_ref, w01_ref, b01_ref, w02_ref, b02_ref),
        (g1_ref, NNBR[1], w10_ref, b10_ref, w11_ref, b11_ref, w12_ref, b12_ref),
    )
    for si, (g_ref, nn, w0_ref, b0_ref, w1_ref, b1_ref, w2_ref, b2_ref) \
            in enumerate(scale_refs):
        g = g_ref[0][:, :, si * CPRE:(si + 1) * CPRE]
        wx = w0_ref[...][:, :3]
        ct = jax.lax.dot_general(
            c3, wx, (((0,), (1,)), ((), ())),
            preferred_element_type=jnp.float32)
        bct = b0_ref[...] - ct
        h1 = jnp.maximum(g + bct[:, None, :], 0.0)
        h1f = h1.reshape(KBLK * nn, CPRE)
        h2 = jnp.maximum(
            jax.lax.dot_general(h1f, w1_ref[...], (((1,), (1,)), ((), ())),
                                preferred_element_type=jnp.float32)
            + b1_ref[...], 0.0)
        h3 = jnp.maximum(
            jax.lax.dot_general(h2, w2_ref[...], (((1,), (1,)), ((), ())),
                                preferred_element_type=jnp.float32)
            + b2_ref[...], 0.0)
        accs.append(jnp.max(h3.reshape(KBLK, nn, 64), axis=1))
    cat = jnp.concatenate(accs, axis=1)
    out = jnp.maximum(
        jax.lax.dot_general(cat, wagg_ref[...], (((1,), (1,)), ((), ())),
                            preferred_element_type=jnp.float32)
        + bagg_ref[...], 0.0)
    out_ref[0] = out


def _mlp_call(g0, g1, sxyz, w00, b00, w01, b01, w02, b02,
              w10, b10, w11, b11, w12, b12, wagg, bagg):
    full = lambda shape: pl.BlockSpec(shape, lambda b, kb: tuple(0 for _ in shape))
    return pl.pallas_call(
        _mlp_body,
        grid=(NB, NSMP // KBLK),
        in_specs=[
            pl.BlockSpec((1, KBLK, NNBR[0], CROW), lambda b, kb: (b, kb, 0, 0)),
            pl.BlockSpec((1, KBLK, NNBR[1], CROW), lambda b, kb: (b, kb, 0, 0)),
            pl.BlockSpec((1, 3, KBLK), lambda b, kb: (b, 0, kb)),
            full((32, 19)), full((1, 32)),
            full((32, 32)), full((1, 32)),
            full((64, 32)), full((1, 64)),
            full((32, 19)), full((1, 32)),
            full((32, 32)), full((1, 32)),
            full((64, 32)), full((1, 64)),
            full((64, 128)), full((1, 64)),
        ],
        out_specs=pl.BlockSpec((1, KBLK, 64), lambda b, kb: (b, kb, 0)),
        out_shape=jax.ShapeDtypeStruct((NB, NSMP, 64), jnp.float32),
    )(g0, g1, sxyz, w00, b00, w01, b01, w02, b02,
      w10, b10, w11, b11, w12, b12, wagg, bagg)


from jax import lax
from jax.experimental.pallas import tpu_sc as plsc

NTILES = 32
KPT = NB * NSMP // NTILES
NCHUNK = NPT // 16


def _bcast_lane(v, lane):
    idx = jnp.full((16, 1), lane, jnp.int32)
    return lax.gather(
        v, idx,
        lax.GatherDimensionNumbers(offset_dims=(), collapsed_slice_dims=(0,),
                                   start_index_map=(0,)),
        (1,), mode=lax.GatherScatterMode.PROMISE_IN_BOUNDS)


def _sc_ballq_gather(sx, sy, sz, xs, ys, zs, pre):
    mesh = plsc.VectorSubcoreMesh(core_axis_name="c", subcore_axis_name="s")

    @functools.partial(
        pl.kernel, mesh=mesh,
        out_type=[
            jax.ShapeDtypeStruct((NB * NSMP * NNBR[0], CROW), jnp.float32),
            jax.ShapeDtypeStruct((NB * NSMP * NNBR[1], CROW), jnp.float32),
        ],
        compiler_params=pltpu.CompilerParams(needs_layout_passes=False),
        scratch_types=[
            pltpu.VMEM((NPT,), jnp.float32),
            pltpu.VMEM((NPT,), jnp.float32),
            pltpu.VMEM((NPT,), jnp.float32),
            pltpu.VMEM((KPT,), jnp.float32),
            pltpu.VMEM((KPT,), jnp.float32),
            pltpu.VMEM((KPT,), jnp.float32),
            pltpu.VMEM((64,), jnp.int32),
            pltpu.VMEM((KPT * NNBR[0] // 128, 128), jnp.int32),
            pltpu.VMEM((KPT * NNBR[1] // 128, 128), jnp.int32),
            pltpu.VMEM((128, CROW), jnp.float32),
            pltpu.SemaphoreType.DMA,
        ],
    )
    def k(sx_h, sy_h, sz_h, xs_h, ys_h, zs_h, pre_h, g0_h, g1_h,
          xs_v, ys_v, zs_v, sx_v, sy_v, sz_v, tmp_v, idx0_v, idx1_v,
          rows_v, sem):
        wid = lax.axis_index("s") * 2 + lax.axis_index("c")
        base_k = pl.multiple_of(wid * KPT, KPT)
        b = wid // (NTILES // NB)
        gbase = pl.multiple_of(b * NPT, NPT)
        pltpu.sync_copy(xs_h.at[pl.ds(gbase, NPT)], xs_v)
        pltpu.sync_copy(ys_h.at[pl.ds(gbase, NPT)], ys_v)
        pltpu.sync_copy(zs_h.at[pl.ds(gbase, NPT)], zs_v)
        pltpu.sync_copy(sx_h.at[pl.ds(base_k, KPT)], sx_v)
        pltpu.sync_copy(sy_h.at[pl.ds(base_k, KPT)], sy_v)
        pltpu.sync_copy(sz_h.at[pl.ds(base_k, KPT)], sz_v)

        iota16 = lax.iota(jnp.int32, 16)

        def scan_centroid(kk, r2, nn, out_idx_v):
            chunk = kk // 16
            lane = kk - chunk * 16
            cxb = _bcast_lane(sx_v[pl.ds(chunk * 16, 16)], lane)
            cyb = _bcast_lane(sy_v[pl.ds(chunk * 16, 16)], lane)
            czb = _bcast_lane(sz_v[pl.ds(chunk * 16, 16)], lane)

            def cond(st):
                j, cur = st
                return jnp.logical_and(cur < nn, j < NCHUNK)

            def body(st):
                j, cur = st
                off = j * 16
                px = xs_v[pl.ds(off, 16)]
                py = ys_v[pl.ds(off, 16)]
                pz = zs_v[pl.ds(off, 16)]
                dx = px - cxb
                dy = py - cyb
                dz = pz - czb
                d2 = dx * dx + dy * dy + dz * dz
                msk = d2 < r2
                plsc.store_compressed(tmp_v.at[pl.ds(cur, 16)],
                                      iota16 + (off + gbase), mask=msk)
                cnt = plsc.all_reduce_population_count(msk)
                return j + 1, cur + cnt[0]

            _, total = lax.while_loop(cond, body, (jnp.int32(0), jnp.int32(0)))

            s0 = tmp_v[pl.ds(0, 16)]
            firstv = _bcast_lane(s0, 0)
            out0 = jnp.where(iota16 < total, s0, firstv)
            flat = kk * nn
            row = flat // 128
            col = flat - row * 128
            out_idx_v[row, pl.ds(col, 16)] = out0
            if nn > 16:
                s1 = tmp_v[pl.ds(16, 16)]
                out1 = jnp.where(iota16 + 16 < total, s1, firstv)
                out_idx_v[row, pl.ds(col + 16, 16)] = out1

        def scan_all(kk, _):
            scan_centroid(kk, RADII2[0], NNBR[0], idx0_v)
            scan_centroid(kk, RADII2[1], NNBR[1], idx1_v)
            return 0

        lax.fori_loop(0, KPT, scan_all, 0)

        def gather_chunks(idx_v, g_h, nn):
            nchunks = KPT * nn // 128

            def one(c, _):
                pltpu.async_copy(pre_h.at[idx_v.at[c]], rows_v, sem).wait()
                row0 = base_k * nn + c * 128
                pltpu.sync_copy(rows_v, g_h.at[pl.ds(row0, 128)])
                return 0

            lax.fori_loop(0, nchunks, one, 0)

        gather_chunks(idx0_v, g0_h, NNBR[0])
        gather_chunks(idx1_v, g1_h, NNBR[1])

    return k(sx, sy, sz, xs, ys, zs, pre)


def _bq_jnp(sampled, points, radius, nn):
    n = points.shape[1]
    d2 = jnp.sum((sampled[:, :, None, :] - points[:, None, :, :]) ** 2, axis=-1)
    mask = d2 < radius
    idx = jnp.where(mask, jnp.arange(n, dtype=jnp.int32)[None, None, :], n)
    idx = jnp.sort(idx, axis=-1)[:, :, :nn]
    first = idx[:, :, :1]
    idx = jnp.where(idx == n, first, idx)
    return jnp.minimum(idx, n - 1)


def kernel(points, features, w0_0, b0_0, w0_1, b0_1, w0_2, b0_2,
           w1_0, b1_0, w1_1, b1_1, w1_2, b1_2, w_agg, b_agg):
    pts_r = points.transpose(0, 2, 1).reshape(NB, 3, NROW, NCOL)
    xs, ys, zs = pts_r[:, 0], pts_r[:, 1], pts_r[:, 2]
    sxyz = _fps_call(xs, ys, zs)
    sampled = sxyz.transpose(0, 2, 1)

    feats_t = features.transpose(0, 2, 1)
    pre = _pre_call(points, feats_t, w0_0, w1_0)

    g0, g1 = _sc_ballq_gather(
        sxyz[:, 0, :].reshape(-1), sxyz[:, 1, :].reshape(-1),
        sxyz[:, 2, :].reshape(-1),
        xs.reshape(-1), ys.reshape(-1), zs.reshape(-1),
        pre.reshape(NB * NPT, CROW))
    g0 = g0.reshape(NB, NSMP, NNBR[0], CROW)
    g1 = g1.reshape(NB, NSMP, NNBR[1], CROW)

    feats_out = _mlp_call(
        g0, g1, sxyz,
        w0_0, b0_0.reshape(1, -1), w0_1, b0_1.reshape(1, -1),
        w0_2, b0_2.reshape(1, -1),
        w1_0, b1_0.reshape(1, -1), w1_1, b1_1.reshape(1, -1),
        w1_2, b1_2.reshape(1, -1),
        w_agg, b_agg.reshape(1, -1))
    return sampled, feats_out.transpose(0, 2, 1)
